# baseline (speedup 1.0000x reference)
import functools

import jax
import jax.numpy as jnp
from jax.experimental import pallas as pl


def _rvq_body(nq, emb_ref, cb_ref, hcsq_ref, out_ref):
    resid = emb_ref[...]
    tn = resid.shape[0]
    k = cb_ref.shape[1]
    for q in range(nq):
        cb = cb_ref[q]
        dots = jax.lax.dot_general(
            resid, cb, (((1,), (1,)), ((), ())),
            preferred_element_type=jnp.float32,
            precision=jax.lax.Precision.DEFAULT,
        )
        score = dots - hcsq_ref[q][None, :]
        idx = jnp.argmax(score, axis=1).astype(jnp.int32)
        out_ref[q, :] = idx
        if q < nq - 1:
            iota = jax.lax.broadcasted_iota(jnp.int32, (tn, k), 1)
            onehot = (iota == idx[:, None]).astype(jnp.float32)
            quant = jax.lax.dot_general(
                onehot, cb, (((1,), (0,)), ((), ())),
                preferred_element_type=jnp.float32,
                precision=jax.lax.Precision.HIGHEST,
            )
            resid = resid - quant


def kernel(embedding, codebooks):
    n, d = embedding.shape
    nq, k, _ = codebooks.shape
    tn = min(512, n)
    grid = n // tn
    half_csq = 0.5 * jnp.sum(codebooks * codebooks, axis=-1)

    out = pl.pallas_call(
        functools.partial(_rvq_body, nq),
        grid=(grid,),
        in_specs=[
            pl.BlockSpec((tn, d), lambda i: (i, 0)),
            pl.BlockSpec((nq, k, d), lambda i: (0, 0, 0)),
            pl.BlockSpec((nq, k), lambda i: (0, 0)),
        ],
        out_specs=pl.BlockSpec((nq, tn), lambda i: (0, i)),
        out_shape=jax.ShapeDtypeStruct((nq, n), jnp.int32),
    )(embedding, codebooks, half_csq)

    return jnp.transpose(out)[None, :, :]


# speedup vs baseline: 3.1383x; 3.1383x over previous
import functools

import jax
import jax.numpy as jnp
from jax.experimental import pallas as pl


def _rvq_body(nq, emb_ref, cb_ref, hcsq_ref, out_ref):
    resid = emb_ref[...]
    tn = resid.shape[0]
    k = cb_ref.shape[1]

    def mm(a, b, contract_b):
        return jax.lax.dot_general(
            a, b, (((1,), (contract_b,)), ((), ())),
            preferred_element_type=jnp.float32,
        )

    for q in range(nq):
        cb = cb_ref[q]
        dots = mm(resid, cb, 1)
        score = dots - hcsq_ref[q][None, :]
        idx = jnp.argmax(score, axis=1).astype(jnp.int32)
        out_ref[q, :] = idx
        if q < nq - 1:
            iota = jax.lax.broadcasted_iota(jnp.int32, (tn, k), 1)
            onehot = (iota == idx[:, None]).astype(jnp.float32)
            quant = mm(onehot, cb, 0)
            resid = resid - quant


def kernel(embedding, codebooks):
    n, d = embedding.shape
    nq, k, _ = codebooks.shape
    tn = min(512, n)
    grid = n // tn
    half_csq = 0.5 * jnp.sum(codebooks * codebooks, axis=-1)

    out = pl.pallas_call(
        functools.partial(_rvq_body, nq),
        grid=(grid,),
        in_specs=[
            pl.BlockSpec((tn, d), lambda i: (i, 0)),
            pl.BlockSpec((nq, k, d), lambda i: (0, 0, 0)),
            pl.BlockSpec((nq, k), lambda i: (0, 0)),
        ],
        out_specs=pl.BlockSpec((nq, tn), lambda i: (0, i)),
        out_shape=jax.ShapeDtypeStruct((nq, n), jnp.int32),
    )(embedding, codebooks, half_csq)

    return jnp.transpose(out)[None, :, :]
